# baseline (speedup 1.0000x reference)
import functools

import jax
import jax.numpy as jnp
from jax import lax
from jax.experimental import pallas as pl
from jax.experimental.pallas import tpu as pltpu
from jax.experimental.pallas import tpu_sc as plsc

BATCH = 4096
SEQ = 200
DIM = 64
TOTAL = BATCH * SEQ
VOCAB_ROWS = 1000002

A_COLS = 256
A_GRID = (VOCAB_ROWS + A_COLS - 1) // A_COLS
PAIR_ROWS = A_GRID * 128


def _pack_body(tin, tout):
    t = tin[...].T
    tout[...] = jnp.concatenate([t[0:128], t[128:256]], axis=1)


def _pack_table(table_t):
    return pl.pallas_call(
        _pack_body,
        grid=(A_GRID,),
        in_specs=[pl.BlockSpec((DIM, A_COLS), lambda j: (0, j))],
        out_specs=pl.BlockSpec((128, 128), lambda j: (j, 0)),
        out_shape=jax.ShapeDtypeStruct((PAIR_ROWS, 128), jnp.float32),
    )(table_t)


NUM_CORES = 2
NUM_SUBCORES = 16
NW = NUM_CORES * NUM_SUBCORES
B_PER_W = TOTAL // NW
NBUF = 4
CHUNK = 160
N_CHUNKS = B_PER_W // CHUNK
LOOKAHEAD = 2
N_OUTER = N_CHUNKS // NBUF

_MESH = plsc.VectorSubcoreMesh(core_axis_name="c", subcore_axis_name="s")

_SCRATCH = (
    [pltpu.VMEM((CHUNK,), jnp.int32) for _ in range(NBUF)]
    + [pltpu.VMEM((CHUNK,), jnp.int32) for _ in range(NBUF)]
    + [pltpu.VMEM((CHUNK, 128), jnp.float32) for _ in range(NBUF)]
    + [pltpu.SemaphoreType.DMA for _ in range(2 * NBUF)]
)


@functools.partial(
    pl.kernel,
    mesh=_MESH,
    out_type=jax.ShapeDtypeStruct((TOTAL, 128), jnp.float32),
    scratch_types=_SCRATCH,
)
def _gather_pairs(tok_hbm, table_hbm, out_hbm, *refs):
    tok_v = refs[0:NBUF]
    idx_v = refs[NBUF : 2 * NBUF]
    rows_v = refs[2 * NBUF : 3 * NBUF]
    sg = refs[3 * NBUF : 4 * NBUF]
    ss = refs[4 * NBUF : 5 * NBUF]

    wid = lax.axis_index("s") * NUM_CORES + lax.axis_index("c")
    base = wid * B_PER_W

    def launch(i, b):
        off = base + i * CHUNK
        pltpu.sync_copy(tok_hbm.at[pl.ds(off, CHUNK)], tok_v[b])
        for k in range(CHUNK // 16):
            t = tok_v[b][pl.ds(16 * k, 16)]
            p = ((t >> 8) << 7) | (t & 127)
            idx_v[b][pl.ds(16 * k, 16)] = p
        pltpu.async_copy(table_hbm.at[idx_v[b]], rows_v[b], sg[b])

    def gather_wait(b):
        pltpu.make_async_copy(table_hbm.at[idx_v[b]], rows_v[b], sg[b]).wait()

    def store_start(i, b):
        off = base + i * CHUNK
        pltpu.async_copy(rows_v[b], out_hbm.at[pl.ds(off, CHUNK)], ss[b])

    def store_wait(b):
        pltpu.make_async_copy(
            rows_v[b], out_hbm.at[pl.ds(base, CHUNK)], ss[b]
        ).wait()

    for i in range(LOOKAHEAD):
        launch(i, i % NBUF)

    for b in range(NBUF):
        gather_wait(b)
        store_start(b, b)
        ni = b + LOOKAHEAD
        nb = ni % NBUF
        if ni < NBUF:
            launch(ni, nb)
        else:
            store_wait(nb)
            launch(ni, nb)

    def outer(g, carry):
        for b in range(NBUF):
            i = g * NBUF + b
            gather_wait(b)
            store_start(i, b)
            ni = i + LOOKAHEAD
            nb = (b + LOOKAHEAD) % NBUF

            @pl.when(ni < N_CHUNKS)
            def _():
                store_wait(nb)
                launch(ni, nb)

        return carry

    lax.fori_loop(1, N_OUTER, outer, 0)

    for b in range(NBUF):
        store_wait(b)


def _unpack_body(rin, tin, gout):
    xt = rin[...].T
    half = ((tin[...] >> 7) & 1) == 1
    hb = lax.broadcast_in_dim(half, (DIM, 128), (1,))
    gout[...] = jnp.where(hb, xt[DIM : 2 * DIM], xt[0:DIM])[None]


def _unpack_out(rows, toks):
    return pl.pallas_call(
        _unpack_body,
        grid=(SEQ, BATCH // 128),
        in_specs=[
            pl.BlockSpec((128, 128), lambda s, bt: (s * (BATCH // 128) + bt, 0)),
            pl.BlockSpec((128,), lambda s, bt: (s * (BATCH // 128) + bt,)),
        ],
        out_specs=pl.BlockSpec((1, DIM, 128), lambda s, bt: (s, 0, bt)),
        out_shape=jax.ShapeDtypeStruct((SEQ, DIM, BATCH), jnp.float32),
    )(rows, toks)


def kernel(token_seqs, emb_table):
    table_t = emb_table.T
    table_pairs = _pack_table(table_t)
    toks = token_seqs.T.reshape(-1).astype(jnp.int32)
    rows = _gather_pairs(toks, table_pairs)
    g = _unpack_out(rows, toks)
    return g.transpose(2, 0, 1)


# speedup vs baseline: 7.4550x; 7.4550x over previous
import functools

import jax
import jax.numpy as jnp
from jax import lax
from jax.experimental import pallas as pl
from jax.experimental.pallas import tpu as pltpu
from jax.experimental.pallas import tpu_sc as plsc

BATCH = 4096
SEQ = 200
DIM = 64
TOTAL = BATCH * SEQ
VOCAB_ROWS = 1000002

A_BLK = 4096
A_HALF = A_BLK // 2
A_GRID = (VOCAB_ROWS + A_BLK - 1) // A_BLK
PACK_ROWS = A_GRID * A_HALF
TABLE_ROWS = 2 * PACK_ROWS


def _pack_body(tin, tout):
    t = tin[...].T
    tout[...] = jnp.concatenate([t[0:A_HALF], t[A_HALF:A_BLK]], axis=1)


def _pack_table(table_t):
    return pl.pallas_call(
        _pack_body,
        grid=(A_GRID,),
        in_specs=[pl.BlockSpec((DIM, A_BLK), lambda j: (0, j))],
        out_specs=pl.BlockSpec((A_HALF, 128), lambda j: (j, 0)),
        out_shape=jax.ShapeDtypeStruct((PACK_ROWS, 128), jnp.float32),
    )(table_t)


NUM_CORES = 2
NUM_SUBCORES = 16
NW = NUM_CORES * NUM_SUBCORES
B_PER_W = TOTAL // NW
NBUF = 4
CHUNK = 256
N_CHUNKS = B_PER_W // CHUNK
LOOKAHEAD = 2
N_OUTER = N_CHUNKS // NBUF
OUT_ROWS = TOTAL // 2

_MESH = plsc.VectorSubcoreMesh(core_axis_name="c", subcore_axis_name="s")

_SCRATCH = (
    [pltpu.VMEM((CHUNK,), jnp.int32) for _ in range(NBUF)]
    + [pltpu.VMEM((CHUNK,), jnp.int32) for _ in range(NBUF)]
    + [pltpu.VMEM((CHUNK, DIM), jnp.float32) for _ in range(NBUF)]
    + [pltpu.SemaphoreType.DMA for _ in range(2 * NBUF)]
)


@functools.partial(
    pl.kernel,
    mesh=_MESH,
    out_type=jax.ShapeDtypeStruct((OUT_ROWS, 128), jnp.float32),
    scratch_types=_SCRATCH,
    compiler_params=pltpu.CompilerParams(use_tc_tiling_on_sc=False),
)
def _gather_rows(tok_hbm, table_hbm, out_hbm, *refs):
    tok_v = refs[0:NBUF]
    idx_v = refs[NBUF : 2 * NBUF]
    rows_v = refs[2 * NBUF : 3 * NBUF]
    sg = refs[3 * NBUF : 4 * NBUF]
    ss = refs[4 * NBUF : 5 * NBUF]

    wid = lax.axis_index("s") * NUM_CORES + lax.axis_index("c")
    base = wid * B_PER_W

    def launch(i, b):
        off = base + i * CHUNK
        pltpu.sync_copy(tok_hbm.at[pl.ds(off, CHUNK)], tok_v[b])
        for k in range(CHUNK // 16):
            t = tok_v[b][pl.ds(16 * k, 16)]
            p = ((t >> 12) << 12) | ((t & 2047) << 1) | ((t >> 11) & 1)
            idx_v[b][pl.ds(16 * k, 16)] = p
        pltpu.async_copy(table_hbm.at[idx_v[b]], rows_v[b], sg[b])

    def gather_wait(b):
        pltpu.make_async_copy(table_hbm.at[idx_v[b]], rows_v[b], sg[b]).wait()

    def store_slot(i):
        t0 = base + i * CHUNK
        q0 = t0 & 2047
        r0 = (t0 >> 11) * 1024 + (q0 & 1023)
        h = q0 >> 10
        return r0, h

    def store_start(i, b):
        r0, h = store_slot(i)
        pltpu.async_copy(
            rows_v[b], out_hbm.at[pl.ds(r0, CHUNK), pl.ds(h * DIM, DIM)], ss[b]
        )

    def store_wait(b):
        pltpu.make_async_copy(
            rows_v[b], out_hbm.at[pl.ds(base // 2, CHUNK), pl.ds(0, DIM)], ss[b]
        ).wait()

    for i in range(LOOKAHEAD):
        launch(i, i % NBUF)

    for b in range(NBUF):
        gather_wait(b)
        store_start(b, b)
        ni = b + LOOKAHEAD
        nb = ni % NBUF
        if ni < NBUF:
            launch(ni, nb)
        else:
            store_wait(nb)
            launch(ni, nb)

    def outer(g, carry):
        for b in range(NBUF):
            i = g * NBUF + b
            gather_wait(b)
            store_start(i, b)
            ni = i + LOOKAHEAD
            nb = (b + LOOKAHEAD) % NBUF

            @pl.when(ni < N_CHUNKS)
            def _():
                store_wait(nb)
                launch(ni, nb)

        return carry

    lax.fori_loop(1, N_OUTER, outer, 0)

    for b in range(NBUF):
        store_wait(b)


def _unpack_body(rin, gout):
    xt = rin[0].T
    gout[...] = jnp.concatenate([xt[0:DIM], xt[DIM : 2 * DIM]], axis=1)[None]


def _unpack_out(rows3):
    return pl.pallas_call(
        _unpack_body,
        grid=(SEQ, 2),
        in_specs=[pl.BlockSpec((1, 1024, 128), lambda s, h: (2 * s + h, 0, 0))],
        out_specs=pl.BlockSpec((1, DIM, 2048), lambda s, h: (s, 0, h)),
        out_shape=jax.ShapeDtypeStruct((SEQ, DIM, BATCH), jnp.float32),
    )(rows3)


def kernel(token_seqs, emb_table):
    table_t = emb_table.T
    table_lin = _pack_table(table_t).reshape(TABLE_ROWS, DIM)
    toks = token_seqs.T.reshape(-1).astype(jnp.int32)
    rows = _gather_rows(toks, table_lin)
    rows3 = rows.reshape(TOTAL // 2048, 1024, 128)
    g = _unpack_out(rows3)
    return g.transpose(2, 0, 1)
